# baseline (speedup 1.0000x reference)
import functools

import jax
import jax.numpy as jnp
from jax import lax
from jax.experimental import pallas as pl
from jax.experimental.pallas import tpu as pltpu
from jax.experimental.pallas import tpu_sc as plsc

_H = 112
_W = 112
_C = 256
_PH = 7
_PW = 7
_SR = 2
_SCALE = 112.0
_KP = 1024
_NW = 32
_RPW = _KP // _NW
_NBINS = _PH * _PW
_OUTR = 56


def _sc_roi_align(table, rois_flat, n_imgs):
    mesh = plsc.VectorSubcoreMesh(core_axis_name="c", subcore_axis_name="s")

    @functools.partial(
        pl.kernel,
        out_type=jax.ShapeDtypeStruct((_KP * _OUTR, _C), jnp.float32),
        mesh=mesh,
        scratch_types=[
            pltpu.VMEM((_RPW * 8 + 16,), jnp.float32),
            pltpu.VMEM((2, _PH + 1, 128), jnp.int32),
            pltpu.VMEM((2, _PH, 128), jnp.float32),
            pltpu.VMEM((3, 112, _C // 2), jnp.int32),
            pltpu.VMEM((_OUTR, _C), jnp.float32),
            pltpu.SemaphoreType.DMA,
            pltpu.SemaphoreType.DMA,
            pltpu.SemaphoreType.DMA,
        ],
    )
    def k(table_hbm, rois_hbm, out_hbm, roi_v, idx_v, w_v, dest_v, acc_v,
          sem0, sem1, sem2):
        wid = lax.axis_index("s") * 2 + lax.axis_index("c")
        base_roi = wid * _RPW
        pltpu.sync_copy(rois_hbm.at[pl.ds(base_roi * 8, _RPW * 8)],
                        roi_v.at[pl.ds(0, _RPW * 8)])

        zpad = jnp.zeros((16,), jnp.float32)

        @pl.loop(0, 16)
        def _zp(cc):
            for rr in range(_NBINS, _OUTR):
                acc_v[rr, pl.ds(cc * 16, 16)] = zpad

        lane = jnp.arange(16, dtype=jnp.int32)
        pw_f = (lane >> 1).astype(jnp.float32)
        ix_f = (lane & 1).astype(jnp.float32)
        xmask = lane < 14

        def build_idx(rr):
            par = rr & 1
            rvec = roi_v[pl.ds(rr * 8, 16)]
            idx_v[par, _PH, pl.ds(0, 16)] = rvec.astype(jnp.int32)
            b = idx_v[par, _PH, pl.ds(0, 16)][0]
            b = jnp.minimum(jnp.maximum(b, 0), n_imgs - 1)
            x1 = rvec[1] * _SCALE
            y1 = rvec[2] * _SCALE
            x2 = rvec[3] * _SCALE
            y2 = rvec[4] * _SCALE
            roi_w = jnp.maximum(x2 - x1, 1.0)
            roi_h = jnp.maximum(y2 - y1, 1.0)
            bin_w = roi_w * (1.0 / _PW)
            bin_h = roi_h * (1.0 / _PH)
            b_off = b * (_H * _W)

            x = x1 + pw_f * bin_w + (ix_f + 0.5) * bin_w * (1.0 / _SR)
            x = jnp.maximum(x, 0.0)
            xl = x.astype(jnp.int32)
            xlc = jnp.minimum(xl, _W - 1)
            xh = jnp.minimum(xl + 1, _W - 1)
            xlc = jnp.maximum(xlc, 0)
            xh = jnp.maximum(xh, 0)
            lx = x - xlc.astype(jnp.float32)
            hx = 1.0 - lx
            wxa = jnp.where(xmask, hx, 0.0)
            wxb = jnp.where(xmask, lx, 0.0)
            col_a = b_off + xlc
            col_b = b_off + xh

            y = y1 + pw_f * bin_h + (ix_f + 0.5) * bin_h * (1.0 / _SR)
            y = jnp.maximum(y, 0.0)
            yl_v = y.astype(jnp.int32)
            ylc_v = jnp.minimum(yl_v, _H - 1)
            yh_v = jnp.minimum(yl_v + 1, _H - 1)
            ylc_v = jnp.maximum(ylc_v, 0)
            yh_v = jnp.maximum(yh_v, 0)
            ly_v = y - ylc_v.astype(jnp.float32)
            hy_v = 1.0 - ly_v

            for ph in range(_PH):
                for iy in range(_SR):
                    ln = ph * 2 + iy
                    ylc = ylc_v[ln]
                    yh = yh_v[ln]
                    ly = ly_v[ln]
                    hy = hy_v[ln]
                    p0 = iy * 2 * 28
                    idx_v[par, ph, pl.ds(p0, 16)] = col_a + ylc * _W
                    idx_v[par, ph, pl.ds(p0 + 14, 16)] = col_b + ylc * _W
                    w_v[par, ph, pl.ds(p0, 16)] = hy * wxa
                    w_v[par, ph, pl.ds(p0 + 14, 16)] = hy * wxb
                    idx_v[par, ph, pl.ds(p0 + 28, 16)] = col_a + yh * _W
                    idx_v[par, ph, pl.ds(p0 + 42, 16)] = col_b + yh * _W
                    w_v[par, ph, pl.ds(p0 + 28, 16)] = ly * wxa
                    w_v[par, ph, pl.ds(p0 + 42, 16)] = ly * wxb

        build_idx(jnp.int32(0))
        pltpu.async_copy(table_hbm.at[idx_v.at[0, 0, pl.ds(0, 112)]],
                         dest_v.at[2], sem2)

        @pl.loop(0, _RPW)
        def _roi(r):
            par = r & 1
            build_idx(r + 1)

            sems = (sem0, sem1)
            pending = None
            for j in range(_PH):
                dbuf = 2 if j == 0 else (j & 1)
                if j + 1 < _PH:
                    nxt = pltpu.async_copy(
                        table_hbm.at[idx_v.at[par, j + 1, pl.ds(0, 112)]],
                        dest_v.at[(j + 1) & 1], sems[(j + 1) & 1])
                else:
                    pltpu.async_copy(
                        table_hbm.at[idx_v.at[1 - par, 0, pl.ds(0, 112)]],
                        dest_v.at[2], sem2)
                if j == 0:
                    pltpu.make_async_copy(
                        table_hbm.at[idx_v.at[par, 0, pl.ds(0, 112)]],
                        dest_v.at[2], sem2).wait()
                else:
                    pending.wait()
                if j + 1 < _PH:
                    pending = nxt

                for pw in range(_PW):
                    rows = []
                    ws = []
                    for slot in range(4):
                        for xc in range(2):
                            p0 = slot * 28 + xc * 14
                            gv = w_v[par, j, pl.ds(p0, 16)]
                            r0 = p0 + 2 * pw
                            rows.append((r0, r0 + 1))
                            ws.append((gv[2 * pw], gv[2 * pw + 1]))

                    @plsc.parallel_loop(0, 8)
                    def _cc(cc, rows=rows, ws=ws, j=j, dbuf=dbuf, pw=pw):
                        col = pl.ds(cc * 16, 16)
                        te = []
                        to = []
                        for (ra, rb), (wa, wb) in zip(rows, ws):
                            via = dest_v[dbuf, ra, col]
                            vib = dest_v[dbuf, rb, col]
                            ea = lax.bitcast_convert_type(
                                via << 16, jnp.float32)
                            eb = lax.bitcast_convert_type(
                                vib << 16, jnp.float32)
                            oa = lax.bitcast_convert_type(
                                via & (-65536), jnp.float32)
                            ob = lax.bitcast_convert_type(
                                vib & (-65536), jnp.float32)
                            te.append(wa * ea + wb * eb)
                            to.append(wa * oa + wb * ob)
                        while len(te) > 1:
                            te = [a + b for a, b in zip(te[::2], te[1::2])]
                            to = [a + b for a, b in zip(to[::2], to[1::2])]
                        acc_v[j * _PW + pw, col] = te[0] * (1.0 / (_SR * _SR))
                        acc_v[j * _PW + pw, pl.ds(128 + cc * 16, 16)] = \
                            to[0] * (1.0 / (_SR * _SR))

            out_row = (base_roi + r) * _OUTR
            pltpu.sync_copy(acc_v, out_hbm.at[pl.ds(out_row, _OUTR)])

        pltpu.make_async_copy(
            table_hbm.at[idx_v.at[_RPW & 1, 0, pl.ds(0, 112)]],
            dest_v.at[2], sem2).wait()

    return k(table, rois_flat)


def kernel(input, rois):
    n, c, h, w = input.shape
    k = rois.shape[0]
    table = input.transpose(0, 2, 3, 1).reshape(n * h * w, c // 2, 2)
    table = lax.bitcast_convert_type(table.astype(jnp.bfloat16), jnp.int32)
    rois_p = jnp.pad(rois, ((0, _KP - k), (0, 3)))
    out_rows = _sc_roi_align(table, rois_p.reshape(-1), n)
    out = out_rows.reshape(_KP, _OUTR, 2, c // 2)[:k, :_NBINS]
    out = out.transpose(0, 1, 3, 2).reshape(k, _PH, _PW, c)
    return out.transpose(0, 3, 1, 2)


# speedup vs baseline: 1.2390x; 1.2390x over previous
import functools

import jax
import jax.numpy as jnp
from jax import lax
from jax.experimental import pallas as pl
from jax.experimental.pallas import tpu as pltpu
from jax.experimental.pallas import tpu_sc as plsc

_H = 112
_W = 112
_C = 256
_PH = 7
_PW = 7
_SR = 2
_SCALE = 112.0
_KP = 1024
_NW = 32
_RPW = _KP // _NW
_NBINS = _PH * _PW
_OUTR = 56


def _sc_roi_align(table, rois_flat, n_imgs):
    mesh = plsc.VectorSubcoreMesh(core_axis_name="c", subcore_axis_name="s")

    @functools.partial(
        pl.kernel,
        out_type=jax.ShapeDtypeStruct((_KP * _OUTR, _C), jnp.float32),
        mesh=mesh,
        scratch_types=[
            pltpu.VMEM((_RPW * 8 + 16,), jnp.float32),
            pltpu.VMEM((2, _PH + 1, 128), jnp.int32),
            pltpu.VMEM((2, _PH, 128), jnp.float32),
            pltpu.VMEM((3, 112, _C), jnp.float32),
            pltpu.VMEM((_OUTR, _C), jnp.float32),
            pltpu.SemaphoreType.DMA,
            pltpu.SemaphoreType.DMA,
            pltpu.SemaphoreType.DMA,
        ],
    )
    def k(table_hbm, rois_hbm, out_hbm, roi_v, idx_v, w_v, dest_v, acc_v,
          sem0, sem1, sem2):
        wid = lax.axis_index("s") * 2 + lax.axis_index("c")
        base_roi = wid * _RPW
        pltpu.sync_copy(rois_hbm.at[pl.ds(base_roi * 8, _RPW * 8)],
                        roi_v.at[pl.ds(0, _RPW * 8)])

        zpad = jnp.zeros((16,), jnp.float32)

        @pl.loop(0, 16)
        def _zp(cc):
            for rr in range(_NBINS, _OUTR):
                acc_v[rr, pl.ds(cc * 16, 16)] = zpad

        lane = jnp.arange(16, dtype=jnp.int32)
        pw_f = (lane >> 1).astype(jnp.float32)
        ix_f = (lane & 1).astype(jnp.float32)
        xmask = lane < 14

        def build_idx(rr):
            par = rr & 1
            rvec = roi_v[pl.ds(rr * 8, 16)]
            idx_v[par, _PH, pl.ds(0, 16)] = rvec.astype(jnp.int32)
            b = idx_v[par, _PH, pl.ds(0, 16)][0]
            b = jnp.minimum(jnp.maximum(b, 0), n_imgs - 1)
            x1 = rvec[1] * _SCALE
            y1 = rvec[2] * _SCALE
            x2 = rvec[3] * _SCALE
            y2 = rvec[4] * _SCALE
            roi_w = jnp.maximum(x2 - x1, 1.0)
            roi_h = jnp.maximum(y2 - y1, 1.0)
            bin_w = roi_w * (1.0 / _PW)
            bin_h = roi_h * (1.0 / _PH)
            b_off = b * (_H * _W)

            x = x1 + pw_f * bin_w + (ix_f + 0.5) * bin_w * (1.0 / _SR)
            x = jnp.maximum(x, 0.0)
            xl = x.astype(jnp.int32)
            xlc = jnp.minimum(xl, _W - 1)
            xh = jnp.minimum(xl + 1, _W - 1)
            xlc = jnp.maximum(xlc, 0)
            xh = jnp.maximum(xh, 0)
            lx = x - xlc.astype(jnp.float32)
            hx = 1.0 - lx
            wxa = jnp.where(xmask, hx, 0.0)
            wxb = jnp.where(xmask, lx, 0.0)
            col_a = b_off + xlc
            col_b = b_off + xh

            y = y1 + pw_f * bin_h + (ix_f + 0.5) * bin_h * (1.0 / _SR)
            y = jnp.maximum(y, 0.0)
            yl_v = y.astype(jnp.int32)
            ylc_v = jnp.minimum(yl_v, _H - 1)
            yh_v = jnp.minimum(yl_v + 1, _H - 1)
            ylc_v = jnp.maximum(ylc_v, 0)
            yh_v = jnp.maximum(yh_v, 0)
            ly_v = y - ylc_v.astype(jnp.float32)
            hy_v = 1.0 - ly_v

            for ph in range(_PH):
                for iy in range(_SR):
                    ln = ph * 2 + iy
                    ylc = ylc_v[ln]
                    yh = yh_v[ln]
                    ly = ly_v[ln]
                    hy = hy_v[ln]
                    p0 = iy * 2 * 28
                    idx_v[par, ph, pl.ds(p0, 16)] = col_a + ylc * _W
                    idx_v[par, ph, pl.ds(p0 + 14, 16)] = col_b + ylc * _W
                    w_v[par, ph, pl.ds(p0, 16)] = hy * wxa
                    w_v[par, ph, pl.ds(p0 + 14, 16)] = hy * wxb
                    idx_v[par, ph, pl.ds(p0 + 28, 16)] = col_a + yh * _W
                    idx_v[par, ph, pl.ds(p0 + 42, 16)] = col_b + yh * _W
                    w_v[par, ph, pl.ds(p0 + 28, 16)] = ly * wxa
                    w_v[par, ph, pl.ds(p0 + 42, 16)] = ly * wxb

        build_idx(jnp.int32(0))
        pltpu.async_copy(table_hbm.at[idx_v.at[0, 0, pl.ds(0, 112)]],
                         dest_v.at[2], sem2)

        @pl.loop(0, _RPW)
        def _roi(r):
            par = r & 1
            build_idx(r + 1)

            sems = (sem0, sem1)
            pending = None
            for j in range(_PH):
                dbuf = 2 if j == 0 else (j & 1)
                if j + 1 < _PH:
                    nxt = pltpu.async_copy(
                        table_hbm.at[idx_v.at[par, j + 1, pl.ds(0, 112)]],
                        dest_v.at[(j + 1) & 1], sems[(j + 1) & 1])
                else:
                    pltpu.async_copy(
                        table_hbm.at[idx_v.at[1 - par, 0, pl.ds(0, 112)]],
                        dest_v.at[2], sem2)
                if j == 0:
                    pltpu.make_async_copy(
                        table_hbm.at[idx_v.at[par, 0, pl.ds(0, 112)]],
                        dest_v.at[2], sem2).wait()
                else:
                    pending.wait()
                if j + 1 < _PH:
                    pending = nxt

                for pw in range(_PW):
                    rows = []
                    ws = []
                    for slot in range(4):
                        for xc in range(2):
                            p0 = slot * 28 + xc * 14
                            gv = w_v[par, j, pl.ds(p0, 16)]
                            r0 = p0 + 2 * pw
                            rows.append((r0, r0 + 1))
                            ws.append((gv[2 * pw], gv[2 * pw + 1]))

                    @plsc.parallel_loop(0, 16, unroll=2)
                    def _cc(cc, rows=rows, ws=ws, j=j, dbuf=dbuf, pw=pw):
                        col = pl.ds(cc * 16, 16)
                        terms = []
                        for (ra, rb), (wa, wb) in zip(rows, ws):
                            terms.append(wa * dest_v[dbuf, ra, col]
                                         + wb * dest_v[dbuf, rb, col])
                        while len(terms) > 1:
                            terms = [a + b for a, b in
                                     zip(terms[::2], terms[1::2])]
                        acc_v[j * _PW + pw, col] = \
                            terms[0] * (1.0 / (_SR * _SR))

            out_row = (base_roi + r) * _OUTR
            pltpu.sync_copy(acc_v, out_hbm.at[pl.ds(out_row, _OUTR)])

        pltpu.make_async_copy(
            table_hbm.at[idx_v.at[_RPW & 1, 0, pl.ds(0, 112)]],
            dest_v.at[2], sem2).wait()

    return k(table, rois_flat)


def kernel(input, rois):
    n, c, h, w = input.shape
    k = rois.shape[0]
    table = input.transpose(0, 2, 3, 1).reshape(n * h * w, c)
    rois_p = jnp.pad(rois, ((0, _KP - k), (0, 3)))
    out_rows = _sc_roi_align(table, rois_p.reshape(-1), n)
    out = out_rows.reshape(_KP, _OUTR, c)[:k, :_NBINS]
    out = out.reshape(k, _PH, _PW, c)
    return out.transpose(0, 3, 1, 2)
